# baseline (speedup 1.0000x reference)
import functools

import jax
import jax.numpy as jnp
from jax import lax
from jax.experimental import pallas as pl
from jax.experimental.pallas import tpu as pltpu
from jax.experimental.pallas import tpu_sc as plsc

N_NODES = 10000
F = 128
FH = F // 2
NEG_SLOPE = 0.2

NC = 2
NS = 16
CH = 128
SLAB = 12
L = 16
NPAD = 10240


def _prolog_body(h_ref, w_ref, a_ref, ha_ref, hb_ref, sl_ref, sr_ref,
                 mbig_ref):
    i = pl.program_id(0)
    hl = lax.dot_general(h_ref[...], w_ref[...], (((1,), (1,)), ((), ())),
                         preferred_element_type=jnp.float32)
    ha_ref[...] = hl[:, :FH]
    hb_ref[...] = hl[:, FH:]
    al = a_ref[0, :F]
    ar = a_ref[0, F:]
    sl = jnp.sum(hl * al[None, :], axis=1, keepdims=True)
    sr = jnp.sum(hl * ar[None, :], axis=1, keepdims=True)
    blk = sl.shape[0]
    sl_ref[...] = jnp.broadcast_to(sl, (blk, L))
    sr_ref[...] = jnp.broadcast_to(sr, (blk, L))
    new = jnp.concatenate([jnp.full((1, 1), jnp.max(sl)),
                           jnp.full((1, 1), jnp.max(sr))], axis=1)
    acc = jnp.where(i == 0, new, jnp.maximum(mbig_ref[...], new))
    t = jnp.sum(acc, axis=1, keepdims=True)
    m = jnp.broadcast_to(jnp.where(t > 0, t, t * NEG_SLOPE), (1, 2))
    mbig_ref[...] = jnp.where(i == pl.num_programs(0) - 1, m, acc)


def _epilog_body(acc_ref, den_ref, out_ref):
    t0 = acc_ref[0, 0] + acc_ref[1, 0]
    t1 = acc_ref[0, 1] + acc_ref[1, 1]
    d = den_ref[0][:, 0:1] + den_ref[1][:, 0:1] + 1e-16
    r = jnp.concatenate([t0, t1], axis=1) / d
    out_ref[...] = jnp.where(r > 0, r, jnp.exp(jnp.minimum(r, 0.0)) - 1.0)


def _make_sc_kernel(n_chunks, e_real):
    e_tile = n_chunks * CH
    npt = NPAD // NS
    mesh = plsc.VectorSubcoreMesh(core_axis_name="c", subcore_axis_name="s")

    @functools.partial(
        pl.kernel,
        out_type=[
            jax.ShapeDtypeStruct((NC, 2, NPAD, FH), jnp.float32),
            jax.ShapeDtypeStruct((NC, NPAD, L), jnp.float32),
        ],
        mesh=mesh,
        compiler_params=pltpu.CompilerParams(needs_layout_passes=False,
                                             use_tc_tiling_on_sc=False),
        scratch_types=[
            pltpu.VMEM((SLAB, CH), jnp.int32),
            pltpu.VMEM((SLAB, CH), jnp.int32),
            pltpu.VMEM((CH, FH), jnp.float32),
            pltpu.VMEM((CH, L), jnp.float32),
            pltpu.VMEM((CH, L), jnp.float32),
            pltpu.VMEM((CH, L), jnp.float32),
            pltpu.VMEM((L,), jnp.float32),
            pltpu.VMEM_SHARED((NPAD, FH), jnp.float32),
            pltpu.VMEM_SHARED((NPAD, L), jnp.float32),
            pltpu.SemaphoreType.DMA,
            pltpu.SemaphoreType.DMA,
            pltpu.SemaphoreType.DMA,
        ],
    )
    def sc_kernel(row_hbm, col_hbm, sl_hbm, sr_hbm, ha_hbm, hb_hbm, m_hbm,
                  acc_out, den_out,
                  row_v, col_v, rows_v, ex_v, el_v, er_v, m_v,
                  acc_sh, den_sh, gsem, lsem, rsem):
        cid = lax.axis_index("c")
        sid = lax.axis_index("s")
        wid = cid * NS + sid
        ebase = wid * e_tile
        nbase = sid * npt

        pltpu.sync_copy(m_hbm, m_v)

        zero16 = jnp.zeros((L,), jnp.float32)
        zero_idx = jnp.zeros((L,), jnp.int32)
        iota16 = lax.iota(jnp.int32, L)

        for half in range(2):
            hl_hbm = ha_hbm if half == 0 else hb_hbm

            def zrow(j, _):
                for f in range(FH // L):
                    rows_v[j, pl.ds(f * L, L)] = zero16
                ex_v[j, :] = zero16
                return 0
            lax.fori_loop(0, CH, zrow, 0)
            for p in range(npt // CH):
                pltpu.sync_copy(rows_v,
                                acc_sh.at[pl.ds(nbase + p * CH, CH), :])
            if half == 0:
                for p in range(npt // CH):
                    pltpu.sync_copy(ex_v,
                                    den_sh.at[pl.ds(nbase + p * CH, CH), :])
            plsc.subcore_barrier()

            big_m = m_v[...]

            def chunk(ci, _):
                li = lax.rem(ci, SLAB)

                @pl.when(li == 0)
                def _():
                    si = lax.div(ci, SLAB)
                    pltpu.sync_copy(row_hbm.at[wid, si], row_v)
                    pltpu.sync_copy(col_hbm.at[wid, si], col_v)

                grow = pltpu.async_copy(hl_hbm.at[col_v.at[li]], rows_v,
                                        gsem)
                gsl = pltpu.async_copy(sl_hbm.at[row_v.at[li]], el_v, lsem)
                gsr = pltpu.async_copy(sr_hbm.at[col_v.at[li]], er_v, rsem)
                gsl.wait()
                gsr.wait()

                for g in range(CH // L):
                    off = g * L
                    el = plsc.load_gather(el_v, [off + iota16, zero_idx])
                    er = plsc.load_gather(er_v, [off + iota16, zero_idx])
                    t = el + er
                    e = jnp.where(t > 0, t, t * NEG_SLOPE)
                    eid = ebase + ci * CH + off + iota16
                    ex = jnp.where(eid < e_real, jnp.exp(e - big_m), 0.0)
                    plsc.store_scatter(ex_v, [off + iota16, zero_idx], ex)

                grow.wait()

                def scale(j, _):
                    s = plsc.load_gather(
                        ex_v, [jnp.full((L,), j, jnp.int32), zero_idx])
                    for f in range(FH // L):
                        fs = pl.ds(f * L, L)
                        rows_v[j, fs] = rows_v[j, fs] * s
                    return 0
                lax.fori_loop(0, CH, scale, 0)

                pltpu.sync_copy(rows_v, acc_sh.at[row_v.at[li]], add=True)
                if half == 0:
                    pltpu.sync_copy(ex_v, den_sh.at[row_v.at[li]], add=True)
                return 0

            lax.fori_loop(0, n_chunks, chunk, 0)
            plsc.subcore_barrier()

            pltpu.sync_copy(acc_sh.at[pl.ds(nbase, npt), :],
                            acc_out.at[cid, half, pl.ds(nbase, npt), :])
            if half == 0:
                pltpu.sync_copy(den_sh.at[pl.ds(nbase, npt), :],
                                den_out.at[cid, pl.ds(nbase, npt), :])

    return sc_kernel


def kernel(h, edge_index, edge_weight, W, a):
    n = h.shape[0]
    e = edge_index.shape[1]
    e_real = e + n
    n_chunks = -(-e_real // (NC * NS * CH))
    n_chunks += (-n_chunks) % SLAB
    e_pad = NC * NS * n_chunks * CH

    loops = jnp.arange(n, dtype=edge_index.dtype)
    row = jnp.concatenate([edge_index[0], loops,
                           jnp.zeros((e_pad - e_real,), edge_index.dtype)])
    col = jnp.concatenate([edge_index[1], loops,
                           jnp.zeros((e_pad - e_real,), edge_index.dtype)])
    row4 = row.reshape(NC * NS, n_chunks // SLAB, SLAB, CH)
    col4 = col.reshape(NC * NS, n_chunks // SLAB, SLAB, CH)

    blk = 1000
    grid = n // blk
    ha, hb, sl, sr, mbig = pl.pallas_call(
        _prolog_body,
        grid=(grid,),
        in_specs=[
            pl.BlockSpec((blk, F), lambda i: (i, 0)),
            pl.BlockSpec((F, F), lambda i: (0, 0)),
            pl.BlockSpec((1, 2 * F), lambda i: (0, 0)),
        ],
        out_specs=[
            pl.BlockSpec((blk, FH), lambda i: (i, 0)),
            pl.BlockSpec((blk, FH), lambda i: (i, 0)),
            pl.BlockSpec((blk, L), lambda i: (i, 0)),
            pl.BlockSpec((blk, L), lambda i: (i, 0)),
            pl.BlockSpec((1, 2), lambda i: (0, 0)),
        ],
        out_shape=[
            jax.ShapeDtypeStruct((n, FH), jnp.float32),
            jax.ShapeDtypeStruct((n, FH), jnp.float32),
            jax.ShapeDtypeStruct((n, L), jnp.float32),
            jax.ShapeDtypeStruct((n, L), jnp.float32),
            jax.ShapeDtypeStruct((1, 2), jnp.float32),
        ],
    )(h, W, a)

    m16 = jnp.broadcast_to(mbig[0, 0], (L,))
    row4, col4, sl, sr, ha, hb, m16 = lax.optimization_barrier(
        (row4, col4, sl, sr, ha, hb, m16))
    sc = _make_sc_kernel(n_chunks, e_real)
    acc, den = sc(row4, col4, sl, sr, ha, hb, m16)

    blk2 = 1024
    out = pl.pallas_call(
        _epilog_body,
        grid=(NPAD // blk2,),
        in_specs=[
            pl.BlockSpec((NC, 2, blk2, FH), lambda i: (0, 0, i, 0)),
            pl.BlockSpec((NC, blk2, L), lambda i: (0, i, 0)),
        ],
        out_specs=pl.BlockSpec((blk2, F), lambda i: (i, 0)),
        out_shape=jax.ShapeDtypeStruct((NPAD, F), jnp.float32),
    )(acc, den)
    return out[:n]


# speedup vs baseline: 1.6008x; 1.6008x over previous
import functools

import jax
import jax.numpy as jnp
from jax import lax
from jax.experimental import pallas as pl
from jax.experimental.pallas import tpu as pltpu
from jax.experimental.pallas import tpu_sc as plsc

N_NODES = 10000
F = 128
FH = F // 2
NEG_SLOPE = 0.2

NC = 2
NS = 16
CH = 128
SLAB = 9
L = 16
NPAD = 10240


def _prolog_body(h_ref, w_ref, a_ref, ha_ref, hb_ref, sl_ref, sr_ref,
                 mbig_ref):
    i = pl.program_id(0)
    hl = lax.dot_general(h_ref[...], w_ref[...], (((1,), (1,)), ((), ())),
                         preferred_element_type=jnp.float32)
    ha_ref[...] = hl[:, :FH]
    hb_ref[...] = hl[:, FH:]
    al = a_ref[0, :F]
    ar = a_ref[0, F:]
    sl = jnp.sum(hl * al[None, :], axis=1, keepdims=True)
    sr = jnp.sum(hl * ar[None, :], axis=1, keepdims=True)
    blk = sl.shape[0]
    sl_ref[...] = jnp.broadcast_to(sl, (blk, L))
    sr_ref[...] = jnp.broadcast_to(sr, (blk, L))
    new = jnp.concatenate([jnp.full((1, 1), jnp.max(sl)),
                           jnp.full((1, 1), jnp.max(sr))], axis=1)
    acc = jnp.where(i == 0, new, jnp.maximum(mbig_ref[...], new))
    t = jnp.sum(acc, axis=1, keepdims=True)
    m = jnp.broadcast_to(jnp.where(t > 0, t, t * NEG_SLOPE), (1, 2))
    mbig_ref[...] = jnp.where(i == pl.num_programs(0) - 1, m, acc)


def _epilog_body(acc_ref, den_ref, out_ref):
    t0 = acc_ref[0, 0] + acc_ref[1, 0]
    t1 = acc_ref[0, 1] + acc_ref[1, 1]
    d = den_ref[0][:, 0:1] + den_ref[1][:, 0:1] + 1e-16
    r = jnp.concatenate([t0, t1], axis=1) / d
    out_ref[...] = jnp.where(r > 0, r, jnp.exp(jnp.minimum(r, 0.0)) - 1.0)


def _make_sc_kernel(n_chunks, e_real):
    e_tile = n_chunks * CH
    npt = NPAD // NS
    mesh = plsc.VectorSubcoreMesh(core_axis_name="c", subcore_axis_name="s")

    @functools.partial(
        pl.kernel,
        out_type=[
            jax.ShapeDtypeStruct((NC, 2, NPAD, FH), jnp.float32),
            jax.ShapeDtypeStruct((NC, NPAD, L), jnp.float32),
        ],
        mesh=mesh,
        compiler_params=pltpu.CompilerParams(needs_layout_passes=False,
                                             use_tc_tiling_on_sc=False),
        scratch_types=[
            pltpu.VMEM((SLAB, CH), jnp.int32),
            pltpu.VMEM((SLAB, CH), jnp.int32),
            pltpu.VMEM((CH, FH), jnp.float32),
            pltpu.VMEM((CH, L), jnp.float32),
            pltpu.VMEM((CH, L), jnp.float32),
            pltpu.VMEM((CH, L), jnp.float32),
            pltpu.VMEM((L,), jnp.float32),
            pltpu.VMEM_SHARED((NPAD, FH), jnp.float32),
            pltpu.VMEM_SHARED((NPAD, L), jnp.float32),
            pltpu.SemaphoreType.DMA,
            pltpu.SemaphoreType.DMA,
            pltpu.SemaphoreType.DMA,
        ],
    )
    def sc_kernel(row_hbm, col_hbm, sl_hbm, sr_hbm, ha_hbm, hb_hbm, m_hbm,
                  acc_out, den_out,
                  row_v, col_v, rows_v, ex_v, el_v, er_v, m_v,
                  acc_sh, den_sh, gsem, lsem, rsem):
        cid = lax.axis_index("c")
        sid = lax.axis_index("s")
        wid = cid * NS + sid
        ebase = wid * e_tile
        nbase = sid * npt

        pltpu.sync_copy(m_hbm, m_v)

        zero16 = jnp.zeros((L,), jnp.float32)
        zero_idx = jnp.zeros((L,), jnp.int32)
        iota16 = lax.iota(jnp.int32, L)

        for half in range(2):
            hl_hbm = ha_hbm if half == 0 else hb_hbm

            def zrow(j, _):
                for f in range(FH // L):
                    rows_v[j, pl.ds(f * L, L)] = zero16
                ex_v[j, :] = zero16
                return 0
            lax.fori_loop(0, CH, zrow, 0)
            for p in range(npt // CH):
                pltpu.sync_copy(rows_v,
                                acc_sh.at[pl.ds(nbase + p * CH, CH), :])
            if half == 0:
                for p in range(npt // CH):
                    pltpu.sync_copy(ex_v,
                                    den_sh.at[pl.ds(nbase + p * CH, CH), :])
            plsc.subcore_barrier()

            big_m = m_v[...]

            def chunk(ci, _):
                li = lax.rem(ci, SLAB)

                @pl.when(li == 0)
                def _():
                    si = lax.div(ci, SLAB)
                    pltpu.sync_copy(row_hbm.at[wid, si], row_v)
                    pltpu.sync_copy(col_hbm.at[wid, si], col_v)

                grow = pltpu.async_copy(hl_hbm.at[col_v.at[li]], rows_v,
                                        gsem)
                gsl = pltpu.async_copy(sl_hbm.at[row_v.at[li]], el_v, lsem)
                gsr = pltpu.async_copy(sr_hbm.at[col_v.at[li]], er_v, rsem)
                gsl.wait()
                gsr.wait()

                for g in range(CH // L):
                    off = g * L
                    el = plsc.load_gather(el_v, [off + iota16, zero_idx])
                    er = plsc.load_gather(er_v, [off + iota16, zero_idx])
                    t = el + er
                    e = jnp.where(t > 0, t, t * NEG_SLOPE)
                    eid = ebase + ci * CH + off + iota16
                    ex = jnp.where(eid < e_real, jnp.exp(e - big_m), 0.0)
                    plsc.store_scatter(ex_v, [off + iota16, zero_idx], ex)

                grow.wait()

                def scale(j, _):
                    s = plsc.load_gather(
                        ex_v, [jnp.full((L,), j, jnp.int32), zero_idx])
                    for f in range(FH // L):
                        fs = pl.ds(f * L, L)
                        rows_v[j, fs] = rows_v[j, fs] * s
                    return 0
                lax.fori_loop(0, CH, scale, 0)

                pltpu.sync_copy(rows_v, acc_sh.at[row_v.at[li]], add=True)
                if half == 0:
                    pltpu.sync_copy(ex_v, den_sh.at[row_v.at[li]], add=True)
                return 0

            lax.fori_loop(0, n_chunks, chunk, 0)
            plsc.subcore_barrier()

            pltpu.sync_copy(acc_sh.at[pl.ds(nbase, npt), :],
                            acc_out.at[cid, half, pl.ds(nbase, npt), :])
            if half == 0:
                pltpu.sync_copy(den_sh.at[pl.ds(nbase, npt), :],
                                den_out.at[cid, pl.ds(nbase, npt), :])

    return sc_kernel


def kernel(h, edge_index, edge_weight, W, a):
    n = h.shape[0]
    e = edge_index.shape[1]
    e_real = e + n
    n_chunks = -(-e_real // (NC * NS * CH))
    n_chunks += (-n_chunks) % SLAB
    e_pad = NC * NS * n_chunks * CH

    loops = jnp.arange(n, dtype=edge_index.dtype)
    row = jnp.concatenate([edge_index[0], loops,
                           jnp.zeros((e_pad - e_real,), edge_index.dtype)])
    col = jnp.concatenate([edge_index[1], loops,
                           jnp.zeros((e_pad - e_real,), edge_index.dtype)])
    row4 = row.reshape(NC * NS, n_chunks // SLAB, SLAB, CH)
    col4 = col.reshape(NC * NS, n_chunks // SLAB, SLAB, CH)

    blk = 1000
    grid = n // blk
    ha, hb, sl, sr, mbig = pl.pallas_call(
        _prolog_body,
        grid=(grid,),
        in_specs=[
            pl.BlockSpec((blk, F), lambda i: (i, 0)),
            pl.BlockSpec((F, F), lambda i: (0, 0)),
            pl.BlockSpec((1, 2 * F), lambda i: (0, 0)),
        ],
        out_specs=[
            pl.BlockSpec((blk, FH), lambda i: (i, 0)),
            pl.BlockSpec((blk, FH), lambda i: (i, 0)),
            pl.BlockSpec((blk, L), lambda i: (i, 0)),
            pl.BlockSpec((blk, L), lambda i: (i, 0)),
            pl.BlockSpec((1, 2), lambda i: (0, 0)),
        ],
        out_shape=[
            jax.ShapeDtypeStruct((n, FH), jnp.float32),
            jax.ShapeDtypeStruct((n, FH), jnp.float32),
            jax.ShapeDtypeStruct((n, L), jnp.float32),
            jax.ShapeDtypeStruct((n, L), jnp.float32),
            jax.ShapeDtypeStruct((1, 2), jnp.float32),
        ],
    )(h, W, a)

    m16 = jnp.broadcast_to(mbig[0, 0], (L,))
    row4, col4, sl, sr, ha, hb, m16 = lax.optimization_barrier(
        (row4, col4, sl, sr, ha, hb, m16))
    sc = _make_sc_kernel(n_chunks, e_real)
    acc, den = sc(row4, col4, sl, sr, ha, hb, m16)

    blk2 = 1024
    out = pl.pallas_call(
        _epilog_body,
        grid=(NPAD // blk2,),
        in_specs=[
            pl.BlockSpec((NC, 2, blk2, FH), lambda i: (0, 0, i, 0)),
            pl.BlockSpec((NC, blk2, L), lambda i: (0, i, 0)),
        ],
        out_specs=pl.BlockSpec((blk2, F), lambda i: (i, 0)),
        out_shape=jax.ShapeDtypeStruct((NPAD, F), jnp.float32),
    )(acc, den)
    return out[:n]


# speedup vs baseline: 2.1038x; 1.3142x over previous
import functools

import jax
import jax.numpy as jnp
from jax import lax
from jax.experimental import pallas as pl
from jax.experimental.pallas import tpu as pltpu
from jax.experimental.pallas import tpu_sc as plsc

N_NODES = 10000
F = 128
FH = F // 2
NEG_SLOPE = 0.2

NC = 2
NS = 16
CH = 128
SLAB = 9
L = 16
NPAD = 10240


def _prolog_body(h_ref, w_ref, a_ref, ha_ref, hb_ref, sl_ref, sr_ref,
                 mbig_ref):
    i = pl.program_id(0)
    hl = lax.dot_general(h_ref[...], w_ref[...], (((1,), (1,)), ((), ())),
                         preferred_element_type=jnp.float32)
    ha_ref[...] = hl[:, :FH]
    hb_ref[...] = hl[:, FH:]
    al = a_ref[0, :F]
    ar = a_ref[0, F:]
    sl = jnp.sum(hl * al[None, :], axis=1, keepdims=True)
    sr = jnp.sum(hl * ar[None, :], axis=1, keepdims=True)
    blk = sl.shape[0]
    sl_ref[...] = jnp.broadcast_to(sl, (blk, L))
    sr_ref[...] = jnp.broadcast_to(sr, (blk, L))
    new = jnp.concatenate([jnp.full((1, 1), jnp.max(sl)),
                           jnp.full((1, 1), jnp.max(sr))], axis=1)
    acc = jnp.where(i == 0, new, jnp.maximum(mbig_ref[...], new))
    t = jnp.sum(acc, axis=1, keepdims=True)
    m = jnp.broadcast_to(jnp.where(t > 0, t, t * NEG_SLOPE), (1, 2))
    mbig_ref[...] = jnp.where(i == pl.num_programs(0) - 1, m, acc)


def _epilog_body(acc_ref, den_ref, out_ref):
    t0 = acc_ref[0, 0] + acc_ref[1, 0]
    t1 = acc_ref[0, 1] + acc_ref[1, 1]
    d = den_ref[0][:, 0:1] + den_ref[1][:, 0:1] + 1e-16
    r = jnp.concatenate([t0, t1], axis=1) / d
    out_ref[...] = jnp.where(r > 0, r, jnp.exp(jnp.minimum(r, 0.0)) - 1.0)


def _make_sc_kernel(n_chunks, e_real):
    e_tile = n_chunks * CH
    npt = NPAD // NS
    mesh = plsc.VectorSubcoreMesh(core_axis_name="c", subcore_axis_name="s")

    @functools.partial(
        pl.kernel,
        out_type=[
            jax.ShapeDtypeStruct((NC, 2, NPAD, FH), jnp.float32),
            jax.ShapeDtypeStruct((NC, NPAD, L), jnp.float32),
        ],
        mesh=mesh,
        compiler_params=pltpu.CompilerParams(needs_layout_passes=False,
                                             use_tc_tiling_on_sc=False),
        scratch_types=[
            pltpu.VMEM((SLAB, CH), jnp.int32),
            pltpu.VMEM((SLAB, CH), jnp.int32),
            pltpu.VMEM((CH, FH), jnp.float32),
            pltpu.VMEM((CH, L), jnp.float32),
            pltpu.VMEM((CH, L), jnp.float32),
            pltpu.VMEM((CH, L), jnp.float32),
            pltpu.VMEM((L,), jnp.float32),
            pltpu.VMEM_SHARED((NPAD, FH), jnp.float32),
            pltpu.VMEM_SHARED((NPAD, L), jnp.float32),
            pltpu.SemaphoreType.DMA,
            pltpu.SemaphoreType.DMA,
            pltpu.SemaphoreType.DMA,
        ],
    )
    def sc_kernel(row_hbm, col_hbm, sl_hbm, sr_hbm, ha_hbm, hb_hbm, m_hbm,
                  acc_out, den_out,
                  row_v, col_v, rows_v, ex_v, el_v, er_v, m_v,
                  acc_sh, den_sh, gsem, lsem, rsem):
        cid = lax.axis_index("c")
        sid = lax.axis_index("s")
        wid = cid * NS + sid
        ebase = wid * e_tile
        nbase = sid * npt

        pltpu.sync_copy(m_hbm, m_v)

        zero16 = jnp.zeros((L,), jnp.float32)
        zero_idx = jnp.zeros((L,), jnp.int32)
        iota16 = lax.iota(jnp.int32, L)

        for half in range(2):
            hl_hbm = ha_hbm if half == 0 else hb_hbm

            def zrow(j, _):
                for f in range(FH // L):
                    rows_v[j, pl.ds(f * L, L)] = zero16
                ex_v[j, :] = zero16
                return 0
            lax.fori_loop(0, CH, zrow, 0)
            for p in range(npt // CH):
                pltpu.sync_copy(rows_v,
                                acc_sh.at[pl.ds(nbase + p * CH, CH), :])
            if half == 0:
                for p in range(npt // CH):
                    pltpu.sync_copy(ex_v,
                                    den_sh.at[pl.ds(nbase + p * CH, CH), :])
            plsc.subcore_barrier()

            big_m = m_v[...]

            def chunk(ci, _):
                li = lax.rem(ci, SLAB)

                @pl.when(li == 0)
                def _():
                    si = lax.div(ci, SLAB)
                    pltpu.sync_copy(row_hbm.at[wid, si], row_v)
                    pltpu.sync_copy(col_hbm.at[wid, si], col_v)

                grow = pltpu.async_copy(hl_hbm.at[col_v.at[li]], rows_v,
                                        gsem)
                gsl = pltpu.async_copy(sl_hbm.at[row_v.at[li]], el_v, lsem)
                gsr = pltpu.async_copy(sr_hbm.at[col_v.at[li]], er_v, rsem)
                gsl.wait()
                gsr.wait()

                for g in range(CH // L):
                    off = g * L
                    el = plsc.load_gather(el_v, [off + iota16, zero_idx])
                    er = plsc.load_gather(er_v, [off + iota16, zero_idx])
                    t = el + er
                    e = jnp.where(t > 0, t, t * NEG_SLOPE)
                    eid = ebase + ci * CH + off + iota16
                    ex = jnp.where(eid < e_real, jnp.exp(e - big_m), 0.0)
                    plsc.store_scatter(ex_v, [off + iota16, zero_idx], ex)

                grow.wait()

                def scale(j, _):
                    s = plsc.load_gather(
                        ex_v, [jnp.full((L,), j, jnp.int32), zero_idx])
                    for f in range(FH // L):
                        fs = pl.ds(f * L, L)
                        rows_v[j, fs] = rows_v[j, fs] * s
                    return 0
                lax.fori_loop(0, CH, scale, 0)

                pltpu.sync_copy(rows_v, acc_sh.at[row_v.at[li]], add=True)
                if half == 0:
                    pltpu.sync_copy(ex_v, den_sh.at[row_v.at[li]], add=True)
                return 0

            lax.fori_loop(0, n_chunks, chunk, 0)
            plsc.subcore_barrier()

            pltpu.sync_copy(acc_sh.at[pl.ds(nbase, npt), :],
                            acc_out.at[cid, half, pl.ds(nbase, npt), :])
            if half == 0:
                pltpu.sync_copy(den_sh.at[pl.ds(nbase, npt), :],
                                den_out.at[cid, pl.ds(nbase, npt), :])

    return sc_kernel


def kernel(h, edge_index, edge_weight, W, a):
    n = h.shape[0]
    e = edge_index.shape[1]
    e_real = e + n
    n_chunks = -(-e_real // (NC * NS * CH))
    n_chunks += (-n_chunks) % SLAB
    e_pad = NC * NS * n_chunks * CH

    loops = jnp.arange(n, dtype=edge_index.dtype)
    pad = jnp.arange(e_pad - e_real, dtype=edge_index.dtype)
    row = jnp.concatenate([edge_index[0], loops, n + pad % (NPAD - n)])
    col = jnp.concatenate([edge_index[1], loops, pad % n])
    row4 = row.reshape(NC * NS, n_chunks // SLAB, SLAB, CH)
    col4 = col.reshape(NC * NS, n_chunks // SLAB, SLAB, CH)

    blk = 1000
    grid = n // blk
    ha, hb, sl, sr, mbig = pl.pallas_call(
        _prolog_body,
        grid=(grid,),
        in_specs=[
            pl.BlockSpec((blk, F), lambda i: (i, 0)),
            pl.BlockSpec((F, F), lambda i: (0, 0)),
            pl.BlockSpec((1, 2 * F), lambda i: (0, 0)),
        ],
        out_specs=[
            pl.BlockSpec((blk, FH), lambda i: (i, 0)),
            pl.BlockSpec((blk, FH), lambda i: (i, 0)),
            pl.BlockSpec((blk, L), lambda i: (i, 0)),
            pl.BlockSpec((blk, L), lambda i: (i, 0)),
            pl.BlockSpec((1, 2), lambda i: (0, 0)),
        ],
        out_shape=[
            jax.ShapeDtypeStruct((n, FH), jnp.float32),
            jax.ShapeDtypeStruct((n, FH), jnp.float32),
            jax.ShapeDtypeStruct((n, L), jnp.float32),
            jax.ShapeDtypeStruct((n, L), jnp.float32),
            jax.ShapeDtypeStruct((1, 2), jnp.float32),
        ],
    )(h, W, a)

    m16 = jnp.broadcast_to(mbig[0, 0], (L,))
    row4, col4, sl, sr, ha, hb, m16 = lax.optimization_barrier(
        (row4, col4, sl, sr, ha, hb, m16))
    sc = _make_sc_kernel(n_chunks, e_real)
    acc, den = sc(row4, col4, sl, sr, ha, hb, m16)

    blk2 = 1024
    out = pl.pallas_call(
        _epilog_body,
        grid=(NPAD // blk2,),
        in_specs=[
            pl.BlockSpec((NC, 2, blk2, FH), lambda i: (0, 0, i, 0)),
            pl.BlockSpec((NC, blk2, L), lambda i: (0, i, 0)),
        ],
        out_specs=pl.BlockSpec((blk2, F), lambda i: (i, 0)),
        out_shape=jax.ShapeDtypeStruct((NPAD, F), jnp.float32),
    )(acc, den)
    return out[:n]


# speedup vs baseline: 2.3787x; 1.1306x over previous
import functools

import jax
import jax.numpy as jnp
from jax import lax
from jax.experimental import pallas as pl
from jax.experimental.pallas import tpu as pltpu
from jax.experimental.pallas import tpu_sc as plsc

N_NODES = 10000
F = 128
FH = F // 2
NEG_SLOPE = 0.2

NC = 2
NS = 16
CH = 128
SLAB = 12
L = 16
NPAD = 10240


def _prolog_body(h_ref, w_ref, a_ref, ha_ref, hb_ref, sl_ref, sr_ref,
                 mbig_ref):
    i = pl.program_id(0)
    hl = lax.dot_general(h_ref[...], w_ref[...], (((1,), (1,)), ((), ())),
                         preferred_element_type=jnp.float32)
    ha_ref[...] = hl[:, :FH]
    hb_ref[...] = hl[:, FH:]
    al = a_ref[0, :F]
    ar = a_ref[0, F:]
    sl = jnp.sum(hl * al[None, :], axis=1, keepdims=True)
    sr = jnp.sum(hl * ar[None, :], axis=1, keepdims=True)
    blk = sl.shape[0]
    sl_ref[...] = jnp.broadcast_to(sl, (blk, L))
    sr_ref[...] = jnp.broadcast_to(sr, (blk, L))
    new = jnp.concatenate([jnp.full((1, 1), jnp.max(sl)),
                           jnp.full((1, 1), jnp.max(sr))], axis=1)
    acc = jnp.where(i == 0, new, jnp.maximum(mbig_ref[...], new))
    t = jnp.sum(acc, axis=1, keepdims=True)
    m = jnp.broadcast_to(jnp.where(t > 0, t, t * NEG_SLOPE), (1, 2))
    mbig_ref[...] = jnp.where(i == pl.num_programs(0) - 1, m, acc)


def _epilog_body(acc_ref, den_ref, out_ref):
    t0 = acc_ref[0, 0] + acc_ref[1, 0]
    t1 = acc_ref[0, 1] + acc_ref[1, 1]
    d = den_ref[0][:, 0:1] + den_ref[1][:, 0:1] + 1e-16
    r = jnp.concatenate([t0, t1], axis=1) / d
    out_ref[...] = jnp.where(r > 0, r, jnp.exp(jnp.minimum(r, 0.0)) - 1.0)


def _make_sc_kernel(n_chunks, e_real):
    e_tile = n_chunks * CH
    npt = NPAD // NS
    mesh = plsc.VectorSubcoreMesh(core_axis_name="c", subcore_axis_name="s")

    @functools.partial(
        pl.kernel,
        out_type=[
            jax.ShapeDtypeStruct((NC, 2, NPAD, FH), jnp.float32),
            jax.ShapeDtypeStruct((NC, NPAD, L), jnp.float32),
        ],
        mesh=mesh,
        compiler_params=pltpu.CompilerParams(needs_layout_passes=False,
                                             use_tc_tiling_on_sc=False),
        scratch_types=[
            pltpu.VMEM((SLAB, CH), jnp.int32),
            pltpu.VMEM((SLAB, CH), jnp.int32),
            pltpu.VMEM((2, CH, FH), jnp.float32),
            pltpu.VMEM((2, CH, L), jnp.float32),
            pltpu.VMEM((CH, L), jnp.float32),
            pltpu.VMEM((CH, L), jnp.float32),
            pltpu.VMEM((L,), jnp.float32),
            pltpu.VMEM_SHARED((NPAD, FH), jnp.float32),
            pltpu.VMEM_SHARED((NPAD, L), jnp.float32),
            pltpu.SemaphoreType.DMA,
            pltpu.SemaphoreType.DMA,
            pltpu.SemaphoreType.DMA,
            pltpu.SemaphoreType.DMA,
            pltpu.SemaphoreType.DMA,
        ],
    )
    def sc_kernel(row_hbm, col_hbm, sl_hbm, sr_hbm, ha_hbm, hb_hbm, m_hbm,
                  acc_out, den_out,
                  row_v, col_v, rows_v, ex_v, el_v, er_v, m_v,
                  acc_sh, den_sh, gsem, lsem, rsem, ssem0, ssem1):
        cid = lax.axis_index("c")
        sid = lax.axis_index("s")
        wid = cid * NS + sid
        ebase = wid * e_tile
        nbase = sid * npt

        pltpu.sync_copy(m_hbm, m_v)

        zero16 = jnp.zeros((L,), jnp.float32)
        zero_idx = jnp.zeros((L,), jnp.int32)
        iota16 = lax.iota(jnp.int32, L)

        for half in range(2):
            hl_hbm = ha_hbm if half == 0 else hb_hbm
            ssems = (ssem0, ssem1)

            def zrow(j, _):
                for f in range(FH // L):
                    rows_v[0, j, pl.ds(f * L, L)] = zero16
                if half == 0:
                    ex_v[0, j, :] = zero16
                    ex_v[1, j, :] = zero16
                return 0
            lax.fori_loop(0, CH, zrow, 0)
            for p in range(npt // CH):
                pltpu.sync_copy(rows_v.at[0],
                                acc_sh.at[pl.ds(nbase + p * CH, CH), :])
            if half == 0:
                for p in range(npt // CH):
                    pltpu.sync_copy(ex_v.at[0],
                                    den_sh.at[pl.ds(nbase + p * CH, CH), :])
            plsc.subcore_barrier()

            big_m = m_v[...]

            def do_chunk(ci, b):
                li = lax.rem(ci, SLAB)

                @pl.when(li == 0)
                def _():
                    si = lax.div(ci, SLAB)
                    pltpu.sync_copy(row_hbm.at[wid, si], row_v)
                    pltpu.sync_copy(col_hbm.at[wid, si], col_v)

                gsl = pltpu.async_copy(sl_hbm.at[row_v.at[li]], el_v, lsem)
                gsr = pltpu.async_copy(sr_hbm.at[col_v.at[li]], er_v, rsem)

                @pl.when(ci >= 2)
                def _():
                    pltpu.make_async_copy(ha_hbm.at[pl.ds(0, CH), :],
                                          rows_v.at[b], ssems[b]).wait()
                    if half == 0:
                        pltpu.make_async_copy(sl_hbm.at[pl.ds(0, CH), :],
                                              ex_v.at[b], ssems[b]).wait()

                grow = pltpu.async_copy(hl_hbm.at[col_v.at[li]],
                                        rows_v.at[b], gsem)
                gsl.wait()
                gsr.wait()

                for g in range(CH // L):
                    off = g * L
                    el = plsc.load_gather(el_v, [off + iota16, zero_idx])
                    er = plsc.load_gather(er_v, [off + iota16, zero_idx])
                    t = el + er
                    e = jnp.where(t > 0, t, t * NEG_SLOPE)
                    eid = ebase + ci * CH + off + iota16
                    ex = jnp.where(eid < e_real, jnp.exp(e - big_m), 0.0)
                    plsc.store_scatter(ex_v.at[b], [off + iota16, zero_idx],
                                       ex)

                grow.wait()

                def scale(j, _):
                    s = plsc.load_gather(
                        ex_v, [jnp.full((L,), b, jnp.int32),
                               jnp.full((L,), j, jnp.int32), zero_idx])
                    for f in range(FH // L):
                        fs = pl.ds(f * L, L)
                        rows_v[b, j, fs] = rows_v[b, j, fs] * s
                    return 0
                lax.fori_loop(0, CH, scale, 0)

                pltpu.async_copy(rows_v.at[b], acc_sh.at[row_v.at[li]],
                                 ssems[b], add=True)
                if half == 0:
                    pltpu.async_copy(ex_v.at[b], den_sh.at[row_v.at[li]],
                                     ssems[b], add=True)

            def pair(cp, _):
                do_chunk(cp * 2, 0)
                do_chunk(cp * 2 + 1, 1)
                return 0

            lax.fori_loop(0, n_chunks // 2, pair, 0)
            for b in range(2):
                pltpu.make_async_copy(ha_hbm.at[pl.ds(0, CH), :],
                                      rows_v.at[b], ssems[b]).wait()
                if half == 0:
                    pltpu.make_async_copy(sl_hbm.at[pl.ds(0, CH), :],
                                          ex_v.at[b], ssems[b]).wait()
            plsc.subcore_barrier()

            pltpu.sync_copy(acc_sh.at[pl.ds(nbase, npt), :],
                            acc_out.at[cid, half, pl.ds(nbase, npt), :])
            if half == 0:
                pltpu.sync_copy(den_sh.at[pl.ds(nbase, npt), :],
                                den_out.at[cid, pl.ds(nbase, npt), :])

    return sc_kernel


def kernel(h, edge_index, edge_weight, W, a):
    n = h.shape[0]
    e = edge_index.shape[1]
    e_real = e + n
    n_chunks = -(-e_real // (NC * NS * CH))
    n_chunks += (-n_chunks) % SLAB
    e_pad = NC * NS * n_chunks * CH

    loops = jnp.arange(n, dtype=edge_index.dtype)
    pad = jnp.arange(e_pad - e_real, dtype=edge_index.dtype)
    row = jnp.concatenate([edge_index[0], loops, n + pad % (NPAD - n)])
    col = jnp.concatenate([edge_index[1], loops, pad % n])
    row4 = row.reshape(NC * NS, n_chunks // SLAB, SLAB, CH)
    col4 = col.reshape(NC * NS, n_chunks // SLAB, SLAB, CH)

    blk = 1000
    grid = n // blk
    ha, hb, sl, sr, mbig = pl.pallas_call(
        _prolog_body,
        grid=(grid,),
        in_specs=[
            pl.BlockSpec((blk, F), lambda i: (i, 0)),
            pl.BlockSpec((F, F), lambda i: (0, 0)),
            pl.BlockSpec((1, 2 * F), lambda i: (0, 0)),
        ],
        out_specs=[
            pl.BlockSpec((blk, FH), lambda i: (i, 0)),
            pl.BlockSpec((blk, FH), lambda i: (i, 0)),
            pl.BlockSpec((blk, L), lambda i: (i, 0)),
            pl.BlockSpec((blk, L), lambda i: (i, 0)),
            pl.BlockSpec((1, 2), lambda i: (0, 0)),
        ],
        out_shape=[
            jax.ShapeDtypeStruct((n, FH), jnp.float32),
            jax.ShapeDtypeStruct((n, FH), jnp.float32),
            jax.ShapeDtypeStruct((n, L), jnp.float32),
            jax.ShapeDtypeStruct((n, L), jnp.float32),
            jax.ShapeDtypeStruct((1, 2), jnp.float32),
        ],
    )(h, W, a)

    m16 = jnp.broadcast_to(mbig[0, 0], (L,))
    row4, col4, sl, sr, ha, hb, m16 = lax.optimization_barrier(
        (row4, col4, sl, sr, ha, hb, m16))
    sc = _make_sc_kernel(n_chunks, e_real)
    acc, den = sc(row4, col4, sl, sr, ha, hb, m16)

    blk2 = 1024
    out = pl.pallas_call(
        _epilog_body,
        grid=(NPAD // blk2,),
        in_specs=[
            pl.BlockSpec((NC, 2, blk2, FH), lambda i: (0, 0, i, 0)),
            pl.BlockSpec((NC, blk2, L), lambda i: (0, i, 0)),
        ],
        out_specs=pl.BlockSpec((blk2, F), lambda i: (i, 0)),
        out_shape=jax.ShapeDtypeStruct((NPAD, F), jnp.float32),
    )(acc, den)
    return out[:n]


# speedup vs baseline: 3.2358x; 1.3603x over previous
import functools

import jax
import jax.numpy as jnp
from jax import lax
from jax.experimental import pallas as pl
from jax.experimental.pallas import tpu as pltpu
from jax.experimental.pallas import tpu_sc as plsc

N_NODES = 10000
F = 128
FH = F // 2
NEG_SLOPE = 0.2

NC = 2
NS = 16
CH = 128
SLAB = 12
L = 16
NPAD = 10240


def _prolog_body(h_ref, w_ref, a_ref, ha_ref, hb_ref, sl_ref, sr_ref,
                 mbig_ref):
    i = pl.program_id(0)
    hl = lax.dot_general(h_ref[...], w_ref[...], (((1,), (1,)), ((), ())),
                         preferred_element_type=jnp.float32)
    ha_ref[...] = hl[:, :FH]
    hb_ref[...] = hl[:, FH:]
    al = a_ref[0, :F]
    ar = a_ref[0, F:]
    sl = jnp.sum(hl * al[None, :], axis=1, keepdims=True)
    sr = jnp.sum(hl * ar[None, :], axis=1, keepdims=True)
    blk = sl.shape[0]
    sl_ref[...] = jnp.broadcast_to(sl, (blk, L))
    sr_ref[...] = jnp.broadcast_to(sr, (blk, L))
    new = jnp.concatenate([jnp.full((1, 1), jnp.max(sl)),
                           jnp.full((1, 1), jnp.max(sr))], axis=1)
    acc = jnp.where(i == 0, new, jnp.maximum(mbig_ref[...], new))
    t = jnp.sum(acc, axis=1, keepdims=True)
    m = jnp.broadcast_to(jnp.where(t > 0, t, t * NEG_SLOPE), (1, 2))
    mbig_ref[...] = jnp.where(i == pl.num_programs(0) - 1, m, acc)


def _epilog_body(acc_ref, den_ref, out_ref):
    t0 = acc_ref[0, 0] + acc_ref[1, 0]
    t1 = acc_ref[0, 1] + acc_ref[1, 1]
    d = den_ref[0][:, 0:1] + den_ref[1][:, 0:1] + 1e-16
    r = jnp.concatenate([t0, t1], axis=1) / d
    out_ref[...] = jnp.where(r > 0, r, jnp.exp(jnp.minimum(r, 0.0)) - 1.0)


def _make_sc_kernel(n_chunks, e_real):
    e_tile = n_chunks * CH
    npt = NPAD // NS
    mesh = plsc.VectorSubcoreMesh(core_axis_name="c", subcore_axis_name="s")

    @functools.partial(
        pl.kernel,
        out_type=[
            jax.ShapeDtypeStruct((NC, 2, NPAD, FH), jnp.float32),
            jax.ShapeDtypeStruct((NC, NPAD, L), jnp.float32),
        ],
        mesh=mesh,
        compiler_params=pltpu.CompilerParams(needs_layout_passes=False,
                                             use_tc_tiling_on_sc=False),
        scratch_types=[
            pltpu.VMEM((2, SLAB, CH), jnp.int32),
            pltpu.VMEM((2, SLAB, CH), jnp.int32),
            pltpu.VMEM((3, CH, FH), jnp.float32),
            pltpu.VMEM((3, CH, L), jnp.float32),
            pltpu.VMEM((3, CH, L), jnp.float32),
            pltpu.VMEM((3, CH, L), jnp.float32),
            pltpu.VMEM((L,), jnp.float32),
            pltpu.VMEM_SHARED((NPAD, FH), jnp.float32),
            pltpu.VMEM_SHARED((NPAD, L), jnp.float32),
            pltpu.SemaphoreType.DMA,
            pltpu.SemaphoreType.DMA,
            pltpu.SemaphoreType.DMA,
            pltpu.SemaphoreType.DMA,
            pltpu.SemaphoreType.DMA,
            pltpu.SemaphoreType.DMA,
        ],
    )
    def sc_kernel(row_hbm, col_hbm, sl_hbm, sr_hbm, ha_hbm, hb_hbm, m_hbm,
                  acc_out, den_out,
                  row_v, col_v, rows_v, ex_v, el_v, er_v, m_v,
                  acc_sh, den_sh, gsem0, gsem1, gsem2, ssem0, ssem1, ssem2):
        cid = lax.axis_index("c")
        sid = lax.axis_index("s")
        wid = cid * NS + sid
        ebase = wid * e_tile
        nbase = sid * npt

        pltpu.sync_copy(m_hbm, m_v)

        zero16 = jnp.zeros((L,), jnp.float32)
        zero_idx = jnp.zeros((L,), jnp.int32)
        iota16 = lax.iota(jnp.int32, L)

        gsems = (gsem0, gsem1, gsem2)
        ssems = (ssem0, ssem1, ssem2)

        def slab_slot(ci):
            return lax.rem(lax.div(ci, SLAB), 2)

        for half in range(2):
            hl_hbm = ha_hbm if half == 0 else hb_hbm

            def zrow(j, _):
                for f in range(FH // L):
                    rows_v[0, j, pl.ds(f * L, L)] = zero16
                if half == 0:
                    for s in range(3):
                        ex_v[s, j, :] = zero16
                return 0
            lax.fori_loop(0, CH, zrow, 0)
            for p in range(npt // CH):
                pltpu.sync_copy(rows_v.at[0],
                                acc_sh.at[pl.ds(nbase + p * CH, CH), :])
            if half == 0:
                for p in range(npt // CH):
                    pltpu.sync_copy(ex_v.at[0],
                                    den_sh.at[pl.ds(nbase + p * CH, CH), :])
            plsc.subcore_barrier()

            big_m = m_v[...]

            def fetch(ci, s):
                sp = slab_slot(ci)
                li = lax.rem(ci, SLAB)
                pltpu.async_copy(sl_hbm.at[row_v.at[sp, li]], el_v.at[s],
                                 gsems[s])
                pltpu.async_copy(sr_hbm.at[col_v.at[sp, li]], er_v.at[s],
                                 gsems[s])
                pltpu.async_copy(hl_hbm.at[col_v.at[sp, li]], rows_v.at[s],
                                 gsems[s])

            def drain_scatter(s):
                pltpu.make_async_copy(ha_hbm.at[pl.ds(0, CH), :],
                                      rows_v.at[s], ssems[s]).wait()
                if half == 0:
                    pltpu.make_async_copy(sl_hbm.at[pl.ds(0, CH), :],
                                          ex_v.at[s], ssems[s]).wait()

            pltpu.sync_copy(row_hbm.at[wid, 0], row_v.at[0])
            pltpu.sync_copy(col_hbm.at[wid, 0], col_v.at[0])
            fetch(0, 0)

            def do_chunk(ci, b):
                bn = (b + 1) % 3

                @pl.when((lax.rem(ci + 1, SLAB) == 0)
                         & (ci + 1 < n_chunks))
                def _():
                    si = lax.div(ci + 1, SLAB)
                    sp = lax.rem(si, 2)
                    pltpu.sync_copy(row_hbm.at[wid, si], row_v.at[sp])
                    pltpu.sync_copy(col_hbm.at[wid, si], col_v.at[sp])

                @pl.when(ci >= 2)
                def _():
                    drain_scatter(bn)

                @pl.when(ci + 1 < n_chunks)
                def _():
                    fetch(ci + 1, bn)

                pltpu.make_async_copy(sl_hbm.at[pl.ds(0, CH), :],
                                      el_v.at[b], gsems[b]).wait()
                pltpu.make_async_copy(sl_hbm.at[pl.ds(0, CH), :],
                                      er_v.at[b], gsems[b]).wait()
                pltpu.make_async_copy(ha_hbm.at[pl.ds(0, CH), :],
                                      rows_v.at[b], gsems[b]).wait()

                for g in range(CH // L):
                    off = g * L
                    el = plsc.load_gather(
                        el_v, [jnp.full((L,), b, jnp.int32), off + iota16,
                               zero_idx])
                    er = plsc.load_gather(
                        er_v, [jnp.full((L,), b, jnp.int32), off + iota16,
                               zero_idx])
                    t = el + er
                    e = jnp.where(t > 0, t, t * NEG_SLOPE)
                    eid = ebase + ci * CH + off + iota16
                    ex = jnp.where(eid < e_real, jnp.exp(e - big_m), 0.0)
                    plsc.store_scatter(ex_v.at[b], [off + iota16, zero_idx],
                                       ex)

                def scale(j, _):
                    s = plsc.load_gather(
                        ex_v, [jnp.full((L,), b, jnp.int32),
                               jnp.full((L,), j, jnp.int32), zero_idx])
                    for f in range(FH // L):
                        fs = pl.ds(f * L, L)
                        rows_v[b, j, fs] = rows_v[b, j, fs] * s
                    return 0
                lax.fori_loop(0, CH, scale, 0)

                sp = slab_slot(ci)
                li = lax.rem(ci, SLAB)
                pltpu.async_copy(rows_v.at[b], acc_sh.at[row_v.at[sp, li]],
                                 ssems[b], add=True)
                if half == 0:
                    pltpu.async_copy(ex_v.at[b], den_sh.at[row_v.at[sp, li]],
                                     ssems[b], add=True)

            def triple(cp, _):
                do_chunk(cp * 3, 0)
                do_chunk(cp * 3 + 1, 1)
                do_chunk(cp * 3 + 2, 2)
                return 0

            lax.fori_loop(0, n_chunks // 3, triple, 0)
            drain_scatter(1)
            drain_scatter(2)
            plsc.subcore_barrier()

            pltpu.sync_copy(acc_sh.at[pl.ds(nbase, npt), :],
                            acc_out.at[cid, half, pl.ds(nbase, npt), :])
            if half == 0:
                pltpu.sync_copy(den_sh.at[pl.ds(nbase, npt), :],
                                den_out.at[cid, pl.ds(nbase, npt), :])

    return sc_kernel


def kernel(h, edge_index, edge_weight, W, a):
    n = h.shape[0]
    e = edge_index.shape[1]
    e_real = e + n
    n_chunks = -(-e_real // (NC * NS * CH))
    n_chunks += (-n_chunks) % SLAB
    e_pad = NC * NS * n_chunks * CH

    loops = jnp.arange(n, dtype=edge_index.dtype)
    pad = jnp.arange(e_pad - e_real, dtype=edge_index.dtype)
    row = jnp.concatenate([edge_index[0], loops, n + pad % (NPAD - n)])
    col = jnp.concatenate([edge_index[1], loops, pad % n])
    row4 = row.reshape(NC * NS, n_chunks // SLAB, SLAB, CH)
    col4 = col.reshape(NC * NS, n_chunks // SLAB, SLAB, CH)

    blk = 1000
    grid = n // blk
    ha, hb, sl, sr, mbig = pl.pallas_call(
        _prolog_body,
        grid=(grid,),
        in_specs=[
            pl.BlockSpec((blk, F), lambda i: (i, 0)),
            pl.BlockSpec((F, F), lambda i: (0, 0)),
            pl.BlockSpec((1, 2 * F), lambda i: (0, 0)),
        ],
        out_specs=[
            pl.BlockSpec((blk, FH), lambda i: (i, 0)),
            pl.BlockSpec((blk, FH), lambda i: (i, 0)),
            pl.BlockSpec((blk, L), lambda i: (i, 0)),
            pl.BlockSpec((blk, L), lambda i: (i, 0)),
            pl.BlockSpec((1, 2), lambda i: (0, 0)),
        ],
        out_shape=[
            jax.ShapeDtypeStruct((n, FH), jnp.float32),
            jax.ShapeDtypeStruct((n, FH), jnp.float32),
            jax.ShapeDtypeStruct((n, L), jnp.float32),
            jax.ShapeDtypeStruct((n, L), jnp.float32),
            jax.ShapeDtypeStruct((1, 2), jnp.float32),
        ],
    )(h, W, a)

    m16 = jnp.broadcast_to(mbig[0, 0], (L,))
    row4, col4, sl, sr, ha, hb, m16 = lax.optimization_barrier(
        (row4, col4, sl, sr, ha, hb, m16))
    sc = _make_sc_kernel(n_chunks, e_real)
    acc, den = sc(row4, col4, sl, sr, ha, hb, m16)

    blk2 = 1024
    out = pl.pallas_call(
        _epilog_body,
        grid=(NPAD // blk2,),
        in_specs=[
            pl.BlockSpec((NC, 2, blk2, FH), lambda i: (0, 0, i, 0)),
            pl.BlockSpec((NC, blk2, L), lambda i: (0, i, 0)),
        ],
        out_specs=pl.BlockSpec((blk2, F), lambda i: (i, 0)),
        out_shape=jax.ShapeDtypeStruct((NPAD, F), jnp.float32),
    )(acc, den)
    return out[:n]


# speedup vs baseline: 3.4320x; 1.0606x over previous
import functools

import jax
import jax.numpy as jnp
from jax import lax
from jax.experimental import pallas as pl
from jax.experimental.pallas import tpu as pltpu
from jax.experimental.pallas import tpu_sc as plsc

N_NODES = 10000
F = 128
FH = F // 2
NEG_SLOPE = 0.2

NC = 2
NS = 16
CH = 128
SLAB = 12
L = 16
NPAD = 10240


def _prolog_body(h_ref, w_ref, a_ref, ha_ref, hb_ref, sl_ref, sr_ref,
                 mbig_ref):
    i = pl.program_id(0)
    hl = lax.dot_general(h_ref[...], w_ref[...], (((1,), (1,)), ((), ())),
                         preferred_element_type=jnp.float32)
    ha_ref[...] = hl[:, :FH]
    hb_ref[...] = hl[:, FH:]
    al = a_ref[0, :F]
    ar = a_ref[0, F:]
    sl = jnp.sum(hl * al[None, :], axis=1, keepdims=True)
    sr = jnp.sum(hl * ar[None, :], axis=1, keepdims=True)
    blk = sl.shape[0]
    sl_ref[...] = jnp.broadcast_to(sl, (blk, L))
    sr_ref[...] = jnp.broadcast_to(sr, (blk, L))
    new = jnp.concatenate([jnp.full((1, 1), jnp.max(sl)),
                           jnp.full((1, 1), jnp.max(sr))], axis=1)
    acc = jnp.where(i == 0, new, jnp.maximum(mbig_ref[...], new))
    t = jnp.sum(acc, axis=1, keepdims=True)
    m = jnp.broadcast_to(jnp.where(t > 0, t, t * NEG_SLOPE), (1, 2))
    mbig_ref[...] = jnp.where(i == pl.num_programs(0) - 1, m, acc)


def _epilog_body(acc_ref, den_ref, out_ref):
    t0 = acc_ref[0, 0] + acc_ref[1, 0]
    t1 = acc_ref[0, 1] + acc_ref[1, 1]
    d = den_ref[0][:, 0:1] + den_ref[1][:, 0:1] + 1e-16
    r = jnp.concatenate([t0, t1], axis=1) / d
    out_ref[...] = jnp.where(r > 0, r, jnp.exp(jnp.minimum(r, 0.0)) - 1.0)


def _make_sc_kernel(n_chunks, e_real):
    e_tile = n_chunks * CH
    npt = NPAD // NS
    mesh = plsc.VectorSubcoreMesh(core_axis_name="c", subcore_axis_name="s")

    @functools.partial(
        pl.kernel,
        out_type=[
            jax.ShapeDtypeStruct((NC, 2, NPAD, FH), jnp.float32),
            jax.ShapeDtypeStruct((NC, NPAD, L), jnp.float32),
        ],
        mesh=mesh,
        compiler_params=pltpu.CompilerParams(needs_layout_passes=False,
                                             use_tc_tiling_on_sc=False),
        scratch_types=[
            pltpu.VMEM((2, SLAB, CH), jnp.int32),
            pltpu.VMEM((2, SLAB, CH), jnp.int32),
            pltpu.VMEM((3, CH, FH), jnp.float32),
            pltpu.VMEM((3, CH, L), jnp.float32),
            pltpu.VMEM((3, CH, L), jnp.float32),
            pltpu.VMEM((3, CH, L), jnp.float32),
            pltpu.VMEM((L,), jnp.float32),
            pltpu.VMEM((n_chunks, CH), jnp.float32),
            pltpu.VMEM_SHARED((NPAD, FH), jnp.float32),
            pltpu.VMEM_SHARED((NPAD, L), jnp.float32),
            pltpu.SemaphoreType.DMA,
            pltpu.SemaphoreType.DMA,
            pltpu.SemaphoreType.DMA,
            pltpu.SemaphoreType.DMA,
            pltpu.SemaphoreType.DMA,
            pltpu.SemaphoreType.DMA,
        ],
    )
    def sc_kernel(row_hbm, col_hbm, sl_hbm, sr_hbm, ha_hbm, hb_hbm, m_hbm,
                  acc_out, den_out,
                  row_v, col_v, rows_v, ex_v, el_v, er_v, m_v, ex_all,
                  acc_sh, den_sh, gsem0, gsem1, gsem2, ssem0, ssem1, ssem2):
        cid = lax.axis_index("c")
        sid = lax.axis_index("s")
        wid = cid * NS + sid
        ebase = wid * e_tile
        nbase = sid * npt

        pltpu.sync_copy(m_hbm, m_v)

        zero16 = jnp.zeros((L,), jnp.float32)
        zero_idx = jnp.zeros((L,), jnp.int32)
        iota16 = lax.iota(jnp.int32, L)

        gsems = (gsem0, gsem1, gsem2)
        ssems = (ssem0, ssem1, ssem2)

        def slab_slot(ci):
            return lax.rem(lax.div(ci, SLAB), 2)

        for half in range(2):
            hl_hbm = ha_hbm if half == 0 else hb_hbm

            def zrow(j, _):
                for f in range(FH // L):
                    rows_v[0, j, pl.ds(f * L, L)] = zero16
                if half == 0:
                    for s in range(3):
                        ex_v[s, j, :] = zero16
                return 0
            lax.fori_loop(0, CH, zrow, 0)
            for p in range(npt // CH):
                pltpu.sync_copy(rows_v.at[0],
                                acc_sh.at[pl.ds(nbase + p * CH, CH), :])
            if half == 0:
                for p in range(npt // CH):
                    pltpu.sync_copy(ex_v.at[0],
                                    den_sh.at[pl.ds(nbase + p * CH, CH), :])
            plsc.subcore_barrier()

            big_m = m_v[...]

            def fetch(ci, s):
                sp = slab_slot(ci)
                li = lax.rem(ci, SLAB)
                if half == 0:
                    pltpu.async_copy(sl_hbm.at[row_v.at[sp, li]],
                                     el_v.at[s], gsems[s])
                    pltpu.async_copy(sr_hbm.at[col_v.at[sp, li]],
                                     er_v.at[s], gsems[s])
                pltpu.async_copy(hl_hbm.at[col_v.at[sp, li]], rows_v.at[s],
                                 gsems[s])

            def drain_scatter(s):
                pltpu.make_async_copy(ha_hbm.at[pl.ds(0, CH), :],
                                      rows_v.at[s], ssems[s]).wait()
                if half == 0:
                    pltpu.make_async_copy(sl_hbm.at[pl.ds(0, CH), :],
                                          ex_v.at[s], ssems[s]).wait()

            pltpu.sync_copy(row_hbm.at[wid, 0], row_v.at[0])
            pltpu.sync_copy(col_hbm.at[wid, 0], col_v.at[0])
            fetch(0, 0)

            def do_chunk(ci, b):
                bn = (b + 1) % 3

                @pl.when((lax.rem(ci + 1, SLAB) == 0)
                         & (ci + 1 < n_chunks))
                def _():
                    si = lax.div(ci + 1, SLAB)
                    sp = lax.rem(si, 2)
                    pltpu.sync_copy(row_hbm.at[wid, si], row_v.at[sp])
                    pltpu.sync_copy(col_hbm.at[wid, si], col_v.at[sp])

                @pl.when(ci >= 2)
                def _():
                    drain_scatter(bn)

                @pl.when(ci + 1 < n_chunks)
                def _():
                    fetch(ci + 1, bn)

                if half == 0:
                    pltpu.make_async_copy(sl_hbm.at[pl.ds(0, CH), :],
                                          el_v.at[b], gsems[b]).wait()
                    pltpu.make_async_copy(sl_hbm.at[pl.ds(0, CH), :],
                                          er_v.at[b], gsems[b]).wait()
                pltpu.make_async_copy(ha_hbm.at[pl.ds(0, CH), :],
                                      rows_v.at[b], gsems[b]).wait()

                if half == 0:
                    for g in range(CH // L):
                        off = g * L
                        el = plsc.load_gather(
                            el_v, [jnp.full((L,), b, jnp.int32),
                                   off + iota16, zero_idx])
                        er = plsc.load_gather(
                            er_v, [jnp.full((L,), b, jnp.int32),
                                   off + iota16, zero_idx])
                        t = el + er
                        e = jnp.where(t > 0, t, t * NEG_SLOPE)
                        eid = ebase + ci * CH + off + iota16
                        ex = jnp.where(eid < e_real, jnp.exp(e - big_m),
                                       0.0)
                        plsc.store_scatter(ex_v.at[b],
                                           [off + iota16, zero_idx], ex)
                        ex_all[ci, pl.ds(off, L)] = ex

                def scale(j, _):
                    if half == 0:
                        s = plsc.load_gather(
                            ex_v, [jnp.full((L,), b, jnp.int32),
                                   jnp.full((L,), j, jnp.int32), zero_idx])
                    else:
                        s = plsc.load_gather(
                            ex_all, [jnp.full((L,), ci, jnp.int32),
                                     jnp.full((L,), j, jnp.int32)])
                    for f in range(FH // L):
                        fs = pl.ds(f * L, L)
                        rows_v[b, j, fs] = rows_v[b, j, fs] * s
                    return 0
                lax.fori_loop(0, CH, scale, 0)

                sp = slab_slot(ci)
                li = lax.rem(ci, SLAB)
                pltpu.async_copy(rows_v.at[b], acc_sh.at[row_v.at[sp, li]],
                                 ssems[b], add=True)
                if half == 0:
                    pltpu.async_copy(ex_v.at[b], den_sh.at[row_v.at[sp, li]],
                                     ssems[b], add=True)

            def triple(cp, _):
                do_chunk(cp * 3, 0)
                do_chunk(cp * 3 + 1, 1)
                do_chunk(cp * 3 + 2, 2)
                return 0

            lax.fori_loop(0, n_chunks // 3, triple, 0)
            drain_scatter(1)
            drain_scatter(2)
            plsc.subcore_barrier()

            pltpu.sync_copy(acc_sh.at[pl.ds(nbase, npt), :],
                            acc_out.at[cid, half, pl.ds(nbase, npt), :])
            if half == 0:
                pltpu.sync_copy(den_sh.at[pl.ds(nbase, npt), :],
                                den_out.at[cid, pl.ds(nbase, npt), :])

    return sc_kernel


def kernel(h, edge_index, edge_weight, W, a):
    n = h.shape[0]
    e = edge_index.shape[1]
    e_real = e + n
    n_chunks = -(-e_real // (NC * NS * CH))
    n_chunks += (-n_chunks) % SLAB
    e_pad = NC * NS * n_chunks * CH

    loops = jnp.arange(n, dtype=edge_index.dtype)
    pad = jnp.arange(e_pad - e_real, dtype=edge_index.dtype)
    row = jnp.concatenate([edge_index[0], loops, n + pad % (NPAD - n)])
    col = jnp.concatenate([edge_index[1], loops, pad % n])
    row4 = row.reshape(NC * NS, n_chunks // SLAB, SLAB, CH)
    col4 = col.reshape(NC * NS, n_chunks // SLAB, SLAB, CH)

    blk = 1000
    grid = n // blk
    ha, hb, sl, sr, mbig = pl.pallas_call(
        _prolog_body,
        grid=(grid,),
        in_specs=[
            pl.BlockSpec((blk, F), lambda i: (i, 0)),
            pl.BlockSpec((F, F), lambda i: (0, 0)),
            pl.BlockSpec((1, 2 * F), lambda i: (0, 0)),
        ],
        out_specs=[
            pl.BlockSpec((blk, FH), lambda i: (i, 0)),
            pl.BlockSpec((blk, FH), lambda i: (i, 0)),
            pl.BlockSpec((blk, L), lambda i: (i, 0)),
            pl.BlockSpec((blk, L), lambda i: (i, 0)),
            pl.BlockSpec((1, 2), lambda i: (0, 0)),
        ],
        out_shape=[
            jax.ShapeDtypeStruct((n, FH), jnp.float32),
            jax.ShapeDtypeStruct((n, FH), jnp.float32),
            jax.ShapeDtypeStruct((n, L), jnp.float32),
            jax.ShapeDtypeStruct((n, L), jnp.float32),
            jax.ShapeDtypeStruct((1, 2), jnp.float32),
        ],
    )(h, W, a)

    m16 = jnp.broadcast_to(mbig[0, 0], (L,))
    row4, col4, sl, sr, ha, hb, m16 = lax.optimization_barrier(
        (row4, col4, sl, sr, ha, hb, m16))
    sc = _make_sc_kernel(n_chunks, e_real)
    acc, den = sc(row4, col4, sl, sr, ha, hb, m16)

    blk2 = 1024
    out = pl.pallas_call(
        _epilog_body,
        grid=(NPAD // blk2,),
        in_specs=[
            pl.BlockSpec((NC, 2, blk2, FH), lambda i: (0, 0, i, 0)),
            pl.BlockSpec((NC, blk2, L), lambda i: (0, i, 0)),
        ],
        out_specs=pl.BlockSpec((blk2, F), lambda i: (i, 0)),
        out_shape=jax.ShapeDtypeStruct((NPAD, F), jnp.float32),
    )(acc, den)
    return out[:n]


# speedup vs baseline: 3.4718x; 1.0116x over previous
import functools

import jax
import jax.numpy as jnp
from jax import lax
from jax.experimental import pallas as pl
from jax.experimental.pallas import tpu as pltpu
from jax.experimental.pallas import tpu_sc as plsc

N_NODES = 10000
F = 128
FH = F // 2
NEG_SLOPE = 0.2

NC = 2
NS = 16
CH = 128
SLAB = 12
L = 16
NPAD = 10240


def _prolog_body(h_ref, w_ref, a_ref, ha_ref, hb_ref, sl_ref, sr_ref,
                 mbig_ref):
    i = pl.program_id(0)
    hl = lax.dot_general(h_ref[...], w_ref[...], (((1,), (1,)), ((), ())),
                         preferred_element_type=jnp.float32)
    ha_ref[...] = hl[:, :FH]
    hb_ref[...] = hl[:, FH:]
    al = a_ref[0, :F]
    ar = a_ref[0, F:]
    sl = jnp.sum(hl * al[None, :], axis=1, keepdims=True)
    sr = jnp.sum(hl * ar[None, :], axis=1, keepdims=True)
    blk = sl.shape[0]
    sl_ref[...] = jnp.broadcast_to(sl, (blk, L))
    sr_ref[...] = jnp.broadcast_to(sr, (blk, L))
    new = jnp.concatenate([jnp.full((1, 1), jnp.max(sl)),
                           jnp.full((1, 1), jnp.max(sr))], axis=1)
    acc = jnp.where(i == 0, new, jnp.maximum(mbig_ref[...], new))
    t = jnp.sum(acc, axis=1, keepdims=True)
    m = jnp.broadcast_to(jnp.where(t > 0, t, t * NEG_SLOPE), (1, 2))
    mbig_ref[...] = jnp.where(i == pl.num_programs(0) - 1, m, acc)


def _epilog_body(acc_ref, den_ref, out_ref):
    t0 = acc_ref[0, 0] + acc_ref[1, 0]
    t1 = acc_ref[0, 1] + acc_ref[1, 1]
    d = den_ref[0][:, 0:1] + den_ref[1][:, 0:1] + 1e-16
    r = jnp.concatenate([t0, t1], axis=1) / d
    out_ref[...] = jnp.where(r > 0, r, jnp.exp(jnp.minimum(r, 0.0)) - 1.0)


def _make_sc_kernel(n_chunks, e_real):
    e_tile = n_chunks * CH
    npt = NPAD // NS
    mesh = plsc.VectorSubcoreMesh(core_axis_name="c", subcore_axis_name="s")

    @functools.partial(
        pl.kernel,
        out_type=[
            jax.ShapeDtypeStruct((NC, 2, NPAD, FH), jnp.float32),
            jax.ShapeDtypeStruct((NC, NPAD, L), jnp.float32),
        ],
        mesh=mesh,
        compiler_params=pltpu.CompilerParams(needs_layout_passes=False,
                                             use_tc_tiling_on_sc=False),
        scratch_types=[
            pltpu.VMEM((2, SLAB, CH), jnp.int32),
            pltpu.VMEM((2, SLAB, CH), jnp.int32),
            pltpu.VMEM((3, CH, FH), jnp.float32),
            pltpu.VMEM((3, CH, L), jnp.float32),
            pltpu.VMEM((3, CH, L), jnp.float32),
            pltpu.VMEM((3, CH, L), jnp.float32),
            pltpu.VMEM((L,), jnp.float32),
            pltpu.VMEM((n_chunks, CH), jnp.float32),
            pltpu.VMEM_SHARED((NPAD, FH), jnp.float32),
            pltpu.VMEM_SHARED((NPAD, L), jnp.float32),
            pltpu.SemaphoreType.DMA,
            pltpu.SemaphoreType.DMA,
            pltpu.SemaphoreType.DMA,
            pltpu.SemaphoreType.DMA,
            pltpu.SemaphoreType.DMA,
            pltpu.SemaphoreType.DMA,
        ],
    )
    def sc_kernel(row_hbm, col_hbm, sl_hbm, sr_hbm, ha_hbm, hb_hbm, m_hbm,
                  acc_out, den_out,
                  row_v, col_v, rows_v, ex_v, el_v, er_v, m_v, ex_all,
                  acc_sh, den_sh, gsem0, gsem1, gsem2, ssem0, ssem1, ssem2):
        cid = lax.axis_index("c")
        sid = lax.axis_index("s")
        wid = cid * NS + sid
        ebase = wid * e_tile
        nbase = sid * npt

        pltpu.sync_copy(m_hbm, m_v)

        zero16 = jnp.zeros((L,), jnp.float32)
        zero_idx = jnp.zeros((L,), jnp.int32)
        iota16 = lax.iota(jnp.int32, L)

        gsems = (gsem0, gsem1, gsem2)
        ssems = (ssem0, ssem1, ssem2)

        def slab_slot(ci):
            return lax.rem(lax.div(ci, SLAB), 2)

        for half in range(2):
            hl_hbm = ha_hbm if half == 0 else hb_hbm

            def zrow(j, _):
                for f in range(FH // L):
                    rows_v[0, j, pl.ds(f * L, L)] = zero16
                if half == 0:
                    for s in range(3):
                        ex_v[s, j, :] = zero16
                return 0
            lax.fori_loop(0, CH, zrow, 0)
            for p in range(npt // CH):
                pltpu.sync_copy(rows_v.at[0],
                                acc_sh.at[pl.ds(nbase + p * CH, CH), :])
            if half == 0:
                for p in range(npt // CH):
                    pltpu.sync_copy(ex_v.at[0],
                                    den_sh.at[pl.ds(nbase + p * CH, CH), :])
            plsc.subcore_barrier()

            big_m = m_v[...]

            def fetch(ci, s):
                sp = slab_slot(ci)
                li = lax.rem(ci, SLAB)
                if half == 0:
                    pltpu.async_copy(sl_hbm.at[row_v.at[sp, li]],
                                     el_v.at[s], gsems[s])
                    pltpu.async_copy(sr_hbm.at[col_v.at[sp, li]],
                                     er_v.at[s], gsems[s])
                pltpu.async_copy(hl_hbm.at[col_v.at[sp, li]], rows_v.at[s],
                                 gsems[s])

            def drain_scatter(s):
                pltpu.make_async_copy(ha_hbm.at[pl.ds(0, CH), :],
                                      rows_v.at[s], ssems[s]).wait()
                if half == 0:
                    pltpu.make_async_copy(sl_hbm.at[pl.ds(0, CH), :],
                                          ex_v.at[s], ssems[s]).wait()

            pltpu.sync_copy(row_hbm.at[wid, 0], row_v.at[0])
            pltpu.sync_copy(col_hbm.at[wid, 0], col_v.at[0])
            fetch(0, 0)

            def do_chunk(ci, b):
                bn = (b + 1) % 3

                @pl.when((lax.rem(ci + 1, SLAB) == 0)
                         & (ci + 1 < n_chunks))
                def _():
                    si = lax.div(ci + 1, SLAB)
                    sp = lax.rem(si, 2)
                    pltpu.sync_copy(row_hbm.at[wid, si], row_v.at[sp])
                    pltpu.sync_copy(col_hbm.at[wid, si], col_v.at[sp])

                @pl.when(ci >= 2)
                def _():
                    drain_scatter(bn)

                @pl.when(ci + 1 < n_chunks)
                def _():
                    fetch(ci + 1, bn)

                if half == 0:
                    pltpu.make_async_copy(sl_hbm.at[pl.ds(0, CH), :],
                                          el_v.at[b], gsems[b]).wait()
                    pltpu.make_async_copy(sl_hbm.at[pl.ds(0, CH), :],
                                          er_v.at[b], gsems[b]).wait()
                pltpu.make_async_copy(ha_hbm.at[pl.ds(0, CH), :],
                                      rows_v.at[b], gsems[b]).wait()

                if half == 0:
                    for g in range(CH // L):
                        off = g * L
                        el = plsc.load_gather(
                            el_v, [jnp.full((L,), b, jnp.int32),
                                   off + iota16, zero_idx])
                        er = plsc.load_gather(
                            er_v, [jnp.full((L,), b, jnp.int32),
                                   off + iota16, zero_idx])
                        t = el + er
                        e = jnp.where(t > 0, t, t * NEG_SLOPE)
                        eid = ebase + ci * CH + off + iota16
                        ex = jnp.where(eid < e_real, jnp.exp(e - big_m),
                                       0.0)
                        plsc.store_scatter(ex_v.at[b],
                                           [off + iota16, zero_idx], ex)
                        ex_all[ci, pl.ds(off, L)] = ex

                def scale(jh, _):
                    for u in range(2):
                        j = jh * 2 + u
                        if half == 0:
                            s = plsc.load_gather(
                                ex_v, [jnp.full((L,), b, jnp.int32),
                                       jnp.full((L,), j, jnp.int32),
                                       zero_idx])
                        else:
                            s = plsc.load_gather(
                                ex_all, [jnp.full((L,), ci, jnp.int32),
                                         jnp.full((L,), j, jnp.int32)])
                        for f in range(FH // L):
                            fs = pl.ds(f * L, L)
                            rows_v[b, j, fs] = rows_v[b, j, fs] * s
                    return 0
                lax.fori_loop(0, CH // 2, scale, 0)

                sp = slab_slot(ci)
                li = lax.rem(ci, SLAB)
                pltpu.async_copy(rows_v.at[b], acc_sh.at[row_v.at[sp, li]],
                                 ssems[b], add=True)
                if half == 0:
                    pltpu.async_copy(ex_v.at[b], den_sh.at[row_v.at[sp, li]],
                                     ssems[b], add=True)

            def triple(cp, _):
                do_chunk(cp * 3, 0)
                do_chunk(cp * 3 + 1, 1)
                do_chunk(cp * 3 + 2, 2)
                return 0

            lax.fori_loop(0, n_chunks // 3, triple, 0)
            drain_scatter(1)
            drain_scatter(2)
            plsc.subcore_barrier()

            pltpu.sync_copy(acc_sh.at[pl.ds(nbase, npt), :],
                            acc_out.at[cid, half, pl.ds(nbase, npt), :])
            if half == 0:
                pltpu.sync_copy(den_sh.at[pl.ds(nbase, npt), :],
                                den_out.at[cid, pl.ds(nbase, npt), :])

    return sc_kernel


def kernel(h, edge_index, edge_weight, W, a):
    n = h.shape[0]
    e = edge_index.shape[1]
    e_real = e + n
    n_chunks = -(-e_real // (NC * NS * CH))
    n_chunks += (-n_chunks) % SLAB
    e_pad = NC * NS * n_chunks * CH

    loops = jnp.arange(n, dtype=edge_index.dtype)
    pad = jnp.arange(e_pad - e_real, dtype=edge_index.dtype)
    row = jnp.concatenate([edge_index[0], loops, n + pad % (NPAD - n)])
    col = jnp.concatenate([edge_index[1], loops, pad % n])
    row4 = row.reshape(NC * NS, n_chunks // SLAB, SLAB, CH)
    col4 = col.reshape(NC * NS, n_chunks // SLAB, SLAB, CH)

    blk = 1000
    grid = n // blk
    ha, hb, sl, sr, mbig = pl.pallas_call(
        _prolog_body,
        grid=(grid,),
        in_specs=[
            pl.BlockSpec((blk, F), lambda i: (i, 0)),
            pl.BlockSpec((F, F), lambda i: (0, 0)),
            pl.BlockSpec((1, 2 * F), lambda i: (0, 0)),
        ],
        out_specs=[
            pl.BlockSpec((blk, FH), lambda i: (i, 0)),
            pl.BlockSpec((blk, FH), lambda i: (i, 0)),
            pl.BlockSpec((blk, L), lambda i: (i, 0)),
            pl.BlockSpec((blk, L), lambda i: (i, 0)),
            pl.BlockSpec((1, 2), lambda i: (0, 0)),
        ],
        out_shape=[
            jax.ShapeDtypeStruct((n, FH), jnp.float32),
            jax.ShapeDtypeStruct((n, FH), jnp.float32),
            jax.ShapeDtypeStruct((n, L), jnp.float32),
            jax.ShapeDtypeStruct((n, L), jnp.float32),
            jax.ShapeDtypeStruct((1, 2), jnp.float32),
        ],
    )(h, W, a)

    m16 = jnp.broadcast_to(mbig[0, 0], (L,))
    row4, col4, sl, sr, ha, hb, m16 = lax.optimization_barrier(
        (row4, col4, sl, sr, ha, hb, m16))
    sc = _make_sc_kernel(n_chunks, e_real)
    acc, den = sc(row4, col4, sl, sr, ha, hb, m16)

    blk2 = 1024
    out = pl.pallas_call(
        _epilog_body,
        grid=(NPAD // blk2,),
        in_specs=[
            pl.BlockSpec((NC, 2, blk2, FH), lambda i: (0, 0, i, 0)),
            pl.BlockSpec((NC, blk2, L), lambda i: (0, i, 0)),
        ],
        out_specs=pl.BlockSpec((blk2, F), lambda i: (i, 0)),
        out_shape=jax.ShapeDtypeStruct((NPAD, F), jnp.float32),
    )(acc, den)
    return out[:n]


# speedup vs baseline: 3.5545x; 1.0238x over previous
import functools

import jax
import jax.numpy as jnp
from jax import lax
from jax.experimental import pallas as pl
from jax.experimental.pallas import tpu as pltpu
from jax.experimental.pallas import tpu_sc as plsc

N_NODES = 10000
F = 128
FH = F // 2
NEG_SLOPE = 0.2

NC = 2
NS = 16
CH = 128
SLAB = 28
L = 16
NPAD = 10240


def _prolog_body(h_ref, w_ref, a_ref, ha_ref, hb_ref, sl_ref, sr_ref,
                 mbig_ref):
    i = pl.program_id(0)
    hl = lax.dot_general(h_ref[...], w_ref[...], (((1,), (1,)), ((), ())),
                         preferred_element_type=jnp.float32)
    ha_ref[...] = hl[:, :FH]
    hb_ref[...] = hl[:, FH:]
    al = a_ref[0, :F]
    ar = a_ref[0, F:]
    sl = jnp.sum(hl * al[None, :], axis=1, keepdims=True)
    sr = jnp.sum(hl * ar[None, :], axis=1, keepdims=True)
    blk = sl.shape[0]
    sl_ref[...] = jnp.broadcast_to(sl, (blk, L))
    sr_ref[...] = jnp.broadcast_to(sr, (blk, L))
    new = jnp.concatenate([jnp.full((1, 1), jnp.max(sl)),
                           jnp.full((1, 1), jnp.max(sr))], axis=1)
    acc = jnp.where(i == 0, new, jnp.maximum(mbig_ref[...], new))
    t = jnp.sum(acc, axis=1, keepdims=True)
    m = jnp.broadcast_to(jnp.where(t > 0, t, t * NEG_SLOPE), (1, 2))
    mbig_ref[...] = jnp.where(i == pl.num_programs(0) - 1, m, acc)


def _epilog_body(acc_ref, den_ref, out_ref):
    t0 = acc_ref[0, 0] + acc_ref[1, 0]
    t1 = acc_ref[0, 1] + acc_ref[1, 1]
    d = den_ref[0][:, 0:1] + den_ref[1][:, 0:1] + 1e-16
    r = jnp.concatenate([t0, t1], axis=1) / d
    out_ref[...] = jnp.where(r > 0, r, jnp.exp(jnp.minimum(r, 0.0)) - 1.0)


def _make_sc_kernel(n_chunks, e_real):
    e_tile = n_chunks * CH
    npt = NPAD // NS
    mesh = plsc.VectorSubcoreMesh(core_axis_name="c", subcore_axis_name="s")

    @functools.partial(
        pl.kernel,
        out_type=[
            jax.ShapeDtypeStruct((NC, 2, NPAD, FH), jnp.float32),
            jax.ShapeDtypeStruct((NC, NPAD, L), jnp.float32),
        ],
        mesh=mesh,
        compiler_params=pltpu.CompilerParams(needs_layout_passes=False,
                                             use_tc_tiling_on_sc=False),
        scratch_types=[
            pltpu.VMEM((2, SLAB, CH), jnp.int32),
            pltpu.VMEM((2, SLAB, CH), jnp.int32),
            pltpu.VMEM((3, CH, FH), jnp.float32),
            pltpu.VMEM((3, CH, L), jnp.float32),
            pltpu.VMEM((3, CH, L), jnp.float32),
            pltpu.VMEM((3, CH, L), jnp.float32),
            pltpu.VMEM((L,), jnp.float32),
            pltpu.VMEM((n_chunks, CH), jnp.float32),
            pltpu.VMEM_SHARED((NPAD, FH), jnp.float32),
            pltpu.VMEM_SHARED((NPAD, L), jnp.float32),
            pltpu.SemaphoreType.DMA,
            pltpu.SemaphoreType.DMA,
            pltpu.SemaphoreType.DMA,
            pltpu.SemaphoreType.DMA,
            pltpu.SemaphoreType.DMA,
            pltpu.SemaphoreType.DMA,
        ],
    )
    def sc_kernel(row_hbm, col_hbm, sl_hbm, sr_hbm, ha_hbm, hb_hbm, m_hbm,
                  acc_out, den_out,
                  row_v, col_v, rows_v, ex_v, el_v, er_v, m_v, ex_all,
                  acc_sh, den_sh, gsem0, gsem1, gsem2, ssem0, ssem1, ssem2):
        cid = lax.axis_index("c")
        sid = lax.axis_index("s")
        wid = cid * NS + sid
        ebase = wid * e_tile
        nbase = sid * npt

        pltpu.sync_copy(m_hbm, m_v)

        zero16 = jnp.zeros((L,), jnp.float32)
        zero_idx = jnp.zeros((L,), jnp.int32)
        iota16 = lax.iota(jnp.int32, L)

        gsems = (gsem0, gsem1, gsem2)
        ssems = (ssem0, ssem1, ssem2)

        def slab_slot(ci):
            return lax.rem(lax.div(ci, SLAB), 2)

        for half in range(2):
            hl_hbm = ha_hbm if half == 0 else hb_hbm

            def zrow(j, _):
                for f in range(FH // L):
                    rows_v[0, j, pl.ds(f * L, L)] = zero16
                if half == 0:
                    for s in range(3):
                        ex_v[s, j, :] = zero16
                return 0
            lax.fori_loop(0, CH, zrow, 0)
            for p in range(npt // CH):
                pltpu.sync_copy(rows_v.at[0],
                                acc_sh.at[pl.ds(nbase + p * CH, CH), :])
            if half == 0:
                for p in range(npt // CH):
                    pltpu.sync_copy(ex_v.at[0],
                                    den_sh.at[pl.ds(nbase + p * CH, CH), :])
            plsc.subcore_barrier()

            big_m = m_v[...]

            def fetch(ci, s):
                sp = slab_slot(ci)
                li = lax.rem(ci, SLAB)
                if half == 0:
                    pltpu.async_copy(sl_hbm.at[row_v.at[sp, li]],
                                     el_v.at[s], gsems[s])
                    pltpu.async_copy(sr_hbm.at[col_v.at[sp, li]],
                                     er_v.at[s], gsems[s])
                pltpu.async_copy(hl_hbm.at[col_v.at[sp, li]], rows_v.at[s],
                                 gsems[s])

            def drain_scatter(s):
                pltpu.make_async_copy(ha_hbm.at[pl.ds(0, CH), :],
                                      rows_v.at[s], ssems[s]).wait()
                if half == 0:
                    pltpu.make_async_copy(sl_hbm.at[pl.ds(0, CH), :],
                                          ex_v.at[s], ssems[s]).wait()

            pltpu.sync_copy(row_hbm.at[wid, 0], row_v.at[0])
            pltpu.sync_copy(col_hbm.at[wid, 0], col_v.at[0])
            fetch(0, 0)

            def do_chunk(ci, b):
                bn = (b + 1) % 3

                @pl.when((lax.rem(ci + 1, SLAB) == 0)
                         & (ci + 1 < n_chunks))
                def _():
                    si = lax.div(ci + 1, SLAB)
                    sp = lax.rem(si, 2)
                    pltpu.sync_copy(row_hbm.at[wid, si], row_v.at[sp])
                    pltpu.sync_copy(col_hbm.at[wid, si], col_v.at[sp])

                @pl.when(ci >= 2)
                def _():
                    drain_scatter(bn)

                @pl.when(ci + 1 < n_chunks)
                def _():
                    fetch(ci + 1, bn)

                if half == 0:
                    pltpu.make_async_copy(sl_hbm.at[pl.ds(0, CH), :],
                                          el_v.at[b], gsems[b]).wait()
                    pltpu.make_async_copy(sl_hbm.at[pl.ds(0, CH), :],
                                          er_v.at[b], gsems[b]).wait()
                pltpu.make_async_copy(ha_hbm.at[pl.ds(0, CH), :],
                                      rows_v.at[b], gsems[b]).wait()

                if half == 0:
                    for g in range(CH // L):
                        off = g * L
                        el = plsc.load_gather(
                            el_v, [jnp.full((L,), b, jnp.int32),
                                   off + iota16, zero_idx])
                        er = plsc.load_gather(
                            er_v, [jnp.full((L,), b, jnp.int32),
                                   off + iota16, zero_idx])
                        t = el + er
                        e = jnp.where(t > 0, t, t * NEG_SLOPE)
                        eid = ebase + ci * CH + off + iota16
                        ex = jnp.where(eid < e_real, jnp.exp(e - big_m),
                                       0.0)
                        plsc.store_scatter(ex_v.at[b],
                                           [off + iota16, zero_idx], ex)
                        ex_all[ci, pl.ds(off, L)] = ex

                def scale(jh, _):
                    for u in range(2):
                        j = jh * 2 + u
                        if half == 0:
                            s = plsc.load_gather(
                                ex_v, [jnp.full((L,), b, jnp.int32),
                                       jnp.full((L,), j, jnp.int32),
                                       zero_idx])
                        else:
                            s = plsc.load_gather(
                                ex_all, [jnp.full((L,), ci, jnp.int32),
                                         jnp.full((L,), j, jnp.int32)])
                        for f in range(FH // L):
                            fs = pl.ds(f * L, L)
                            rows_v[b, j, fs] = rows_v[b, j, fs] * s
                    return 0
                lax.fori_loop(0, CH // 2, scale, 0)

                sp = slab_slot(ci)
                li = lax.rem(ci, SLAB)
                pltpu.async_copy(rows_v.at[b], acc_sh.at[row_v.at[sp, li]],
                                 ssems[b], add=True)
                if half == 0:
                    pltpu.async_copy(ex_v.at[b], den_sh.at[row_v.at[sp, li]],
                                     ssems[b], add=True)

            def triple(cp, _):
                do_chunk(cp * 3, 0)
                do_chunk(cp * 3 + 1, 1)
                do_chunk(cp * 3 + 2, 2)
                return 0

            lax.fori_loop(0, n_chunks // 3, triple, 0)
            drain_scatter(1)
            drain_scatter(2)
            plsc.subcore_barrier()

            pltpu.sync_copy(acc_sh.at[pl.ds(nbase, npt), :],
                            acc_out.at[cid, half, pl.ds(nbase, npt), :])
            if half == 0:
                pltpu.sync_copy(den_sh.at[pl.ds(nbase, npt), :],
                                den_out.at[cid, pl.ds(nbase, npt), :])

    return sc_kernel


def kernel(h, edge_index, edge_weight, W, a):
    n = h.shape[0]
    e = edge_index.shape[1]
    e_real = e + n
    n_chunks = -(-e_real // (NC * NS * CH))
    n_chunks += (-n_chunks) % SLAB
    e_pad = NC * NS * n_chunks * CH

    loops = jnp.arange(n, dtype=edge_index.dtype)
    pad = jnp.arange(e_pad - e_real, dtype=edge_index.dtype)
    row = jnp.concatenate([edge_index[0], loops, n + pad % (NPAD - n)])
    col = jnp.concatenate([edge_index[1], loops, pad % n])
    row4 = row.reshape(NC * NS, n_chunks // SLAB, SLAB, CH)
    col4 = col.reshape(NC * NS, n_chunks // SLAB, SLAB, CH)

    blk = 1000
    grid = n // blk
    ha, hb, sl, sr, mbig = pl.pallas_call(
        _prolog_body,
        grid=(grid,),
        in_specs=[
            pl.BlockSpec((blk, F), lambda i: (i, 0)),
            pl.BlockSpec((F, F), lambda i: (0, 0)),
            pl.BlockSpec((1, 2 * F), lambda i: (0, 0)),
        ],
        out_specs=[
            pl.BlockSpec((blk, FH), lambda i: (i, 0)),
            pl.BlockSpec((blk, FH), lambda i: (i, 0)),
            pl.BlockSpec((blk, L), lambda i: (i, 0)),
            pl.BlockSpec((blk, L), lambda i: (i, 0)),
            pl.BlockSpec((1, 2), lambda i: (0, 0)),
        ],
        out_shape=[
            jax.ShapeDtypeStruct((n, FH), jnp.float32),
            jax.ShapeDtypeStruct((n, FH), jnp.float32),
            jax.ShapeDtypeStruct((n, L), jnp.float32),
            jax.ShapeDtypeStruct((n, L), jnp.float32),
            jax.ShapeDtypeStruct((1, 2), jnp.float32),
        ],
    )(h, W, a)

    m16 = jnp.broadcast_to(mbig[0, 0], (L,))
    row4, col4, sl, sr, ha, hb, m16 = lax.optimization_barrier(
        (row4, col4, sl, sr, ha, hb, m16))
    sc = _make_sc_kernel(n_chunks, e_real)
    acc, den = sc(row4, col4, sl, sr, ha, hb, m16)

    blk2 = 1024
    out = pl.pallas_call(
        _epilog_body,
        grid=(NPAD // blk2,),
        in_specs=[
            pl.BlockSpec((NC, 2, blk2, FH), lambda i: (0, 0, i, 0)),
            pl.BlockSpec((NC, blk2, L), lambda i: (0, i, 0)),
        ],
        out_specs=pl.BlockSpec((blk2, F), lambda i: (i, 0)),
        out_shape=jax.ShapeDtypeStruct((NPAD, F), jnp.float32),
    )(acc, den)
    return out[:n]
